# Initial kernel scaffold; baseline (speedup 1.0000x reference)
import functools

import jax
import jax.numpy as jnp
from jax import lax
from jax.experimental import pallas as pl
from jax.experimental.pallas import tpu as pltpu
from jax.experimental.pallas import tpu_sc as plsc

NC = 2
NS = 16
NW = NC * NS
C = 128


def _rsqrt16(d):
    i = plsc.bitcast(d, jnp.int32)
    i = jnp.int32(0x5F3759DF) - jnp.right_shift(i, 1)
    y = plsc.bitcast(i, jnp.float32)
    hd = 0.5 * d
    for _ in range(3):
        y = y * (1.5 - hd * y * y)
    return y


def _make_deg_kernel(npad, chunks, seg):
    mesh = plsc.VectorSubcoreMesh(core_axis_name="c", subcore_axis_name="s")

    @functools.partial(
        pl.kernel,
        out_type=jax.ShapeDtypeStruct((NC, npad), jnp.float32),
        mesh=mesh,
        scratch_types=[
            pltpu.VMEM((chunks, C), jnp.int32),
            pltpu.VMEM((chunks, C), jnp.float32),
            pltpu.VMEM((seg,), jnp.float32),
            pltpu.VMEM_SHARED((npad,), jnp.float32),
        ],
    )
    def deg_kernel(col_hbm, ew_hbm, deg_out, col_v, ew_v, zero_v, deg_sh):
        cid = lax.axis_index("c")
        sid = lax.axis_index("s")
        wid = cid * NS + sid
        pltpu.sync_copy(col_hbm.at[wid], col_v)
        pltpu.sync_copy(ew_hbm.at[wid], ew_v)

        def zbody(i, carry):
            zero_v[pl.ds(i * 16, 16)] = jnp.zeros((16,), jnp.float32)
            return carry

        lax.fori_loop(0, seg // 16, zbody, 0)
        pltpu.sync_copy(zero_v, deg_sh.at[pl.ds(sid * seg, seg)])
        plsc.subcore_barrier()

        def body(j, carry):
            pltpu.sync_copy(ew_v.at[j], deg_sh.at[col_v.at[j]], add=True)
            return carry

        lax.fori_loop(0, chunks, body, 0)
        plsc.subcore_barrier()
        pltpu.sync_copy(deg_sh.at[pl.ds(sid * seg, seg)],
                        deg_out.at[cid, pl.ds(sid * seg, seg)])

    return deg_kernel


def _make_gcn_kernel(npad, chunks, seg, d_out):
    mesh = plsc.VectorSubcoreMesh(core_axis_name="c", subcore_axis_name="s")

    @functools.partial(
        pl.kernel,
        out_type=jax.ShapeDtypeStruct((NC, npad, d_out), jnp.float32),
        mesh=mesh,
        scratch_types=[
            pltpu.VMEM((chunks, C), jnp.int32),
            pltpu.VMEM((chunks, C), jnp.int32),
            pltpu.VMEM((chunks, C), jnp.float32),
            pltpu.VMEM((NC, npad), jnp.float32),
            pltpu.VMEM((npad,), jnp.float32),
            pltpu.VMEM((C, d_out), jnp.float32),
            pltpu.VMEM((C,), jnp.float32),
            pltpu.VMEM_SHARED((npad, d_out), jnp.float32),
        ],
    )
    def gcn_kernel(row_hbm, col_hbm, ew_hbm, h_hbm, deg_hbm, s_out,
                   row_v, col_v, ew_v, deg_v, dinv_v, rows_v, coef_v, acc_sh):
        cid = lax.axis_index("c")
        sid = lax.axis_index("s")
        wid = cid * NS + sid
        pltpu.sync_copy(row_hbm.at[wid], row_v)
        pltpu.sync_copy(col_hbm.at[wid], col_v)
        pltpu.sync_copy(ew_hbm.at[wid], ew_v)
        pltpu.sync_copy(deg_hbm, deg_v)

        def dbody(i, carry):
            sl = pl.ds(i * 16, 16)
            d = deg_v[0, sl] + deg_v[1, sl]
            dinv_v[sl] = _rsqrt16(d)
            return carry

        lax.fori_loop(0, npad // 16, dbody, 0)

        def zbody(r, carry):
            for cb in range(d_out // 16):
                rows_v[r, pl.ds(cb * 16, 16)] = jnp.zeros((16,), jnp.float32)
            return carry

        lax.fori_loop(0, C, zbody, 0)
        for t in range(seg // C):
            pltpu.sync_copy(rows_v, acc_sh.at[pl.ds(sid * seg + t * C, C)])
        plsc.subcore_barrier()

        def chunk(j, carry):
            pltpu.sync_copy(h_hbm.at[row_v.at[j]], rows_v)
            for k in range(C // 16):
                sl = pl.ds(k * 16, 16)
                r16 = row_v[j, sl]
                c16 = col_v[j, sl]
                w16 = ew_v[j, sl]
                dr = plsc.load_gather(dinv_v, [r16])
                dc = plsc.load_gather(dinv_v, [c16])
                coef_v[sl] = w16 * dr * dc

            def scale(r, carry2):
                s = coef_v[r]
                for cb in range(d_out // 16):
                    sl2 = pl.ds(cb * 16, 16)
                    rows_v[r, sl2] = rows_v[r, sl2] * s
                return carry2

            lax.fori_loop(0, C, scale, 0)
            pltpu.sync_copy(rows_v, acc_sh.at[col_v.at[j]], add=True)
            return carry

        lax.fori_loop(0, chunks, chunk, 0)
        plsc.subcore_barrier()
        pltpu.sync_copy(acc_sh.at[pl.ds(sid * seg, seg)],
                        s_out.at[cid, pl.ds(sid * seg, seg)])

    return gcn_kernel


def _matmul(x, W):
    n, d_in = x.shape
    d_out = W.shape[1]
    bs = 1000 if n % 1000 == 0 else n

    def body(x_ref, w_ref, o_ref):
        o_ref[...] = jnp.dot(x_ref[...], w_ref[...],
                             preferred_element_type=jnp.float32)

    return pl.pallas_call(
        body,
        grid=(n // bs,),
        in_specs=[
            pl.BlockSpec((bs, d_in), lambda i: (i, 0)),
            pl.BlockSpec((d_in, d_out), lambda i: (0, 0)),
        ],
        out_specs=pl.BlockSpec((bs, d_out), lambda i: (i, 0)),
        out_shape=jax.ShapeDtypeStruct((n, d_out), jnp.float32),
    )(x, W)


def _combine(S, b2):
    _, n, d_out = S.shape
    bs = 1000 if n % 1000 == 0 else n

    def body(s_ref, b_ref, o_ref):
        o_ref[...] = s_ref[0] + s_ref[1] + b_ref[...]

    return pl.pallas_call(
        body,
        grid=(n // bs,),
        in_specs=[
            pl.BlockSpec((NC, bs, d_out), lambda i: (0, i, 0)),
            pl.BlockSpec((1, d_out), lambda i: (0, 0)),
        ],
        out_specs=pl.BlockSpec((bs, d_out), lambda i: (i, 0)),
        out_shape=jax.ShapeDtypeStruct((n, d_out), jnp.float32),
    )(S, b2)


def kernel(x, edge_index, edge_weight, W, b):
    n = x.shape[0]
    e = edge_index.shape[1]
    d_out = W.shape[1]

    row = edge_index[0].astype(jnp.int32)
    col = edge_index[1].astype(jnp.int32)
    loop = jnp.arange(n, dtype=jnp.int32)
    rows = jnp.concatenate([row, loop])
    cols = jnp.concatenate([col, loop])
    ews = jnp.concatenate([edge_weight.astype(jnp.float32),
                           jnp.ones((n,), jnp.float32)])

    per = NW * C
    chunks = -(-(e + n) // per)
    epad = per * chunks
    pad = epad - (e + n)
    rows3 = jnp.concatenate([rows, jnp.zeros((pad,), jnp.int32)]
                            ).reshape(NW, chunks, C)
    cols3 = jnp.concatenate([cols, jnp.zeros((pad,), jnp.int32)]
                            ).reshape(NW, chunks, C)
    ews3 = jnp.concatenate([ews, jnp.zeros((pad,), jnp.float32)]
                           ).reshape(NW, chunks, C)

    seg = ((n + NS * C - 1) // (NS * C)) * C
    npad = NS * seg

    deg = _make_deg_kernel(npad, chunks, seg)(cols3, ews3)
    h = _matmul(x, W)
    S = _make_gcn_kernel(npad, chunks, seg, d_out)(rows3, cols3, ews3, h, deg)
    out = _combine(S[:, :n, :], b.reshape(1, d_out).astype(jnp.float32))
    return out


# speedup vs baseline: 17.4573x; 17.4573x over previous
import functools

import jax
import jax.numpy as jnp
from jax import lax
from jax.experimental import pallas as pl
from jax.experimental.pallas import tpu as pltpu
from jax.experimental.pallas import tpu_sc as plsc

NC = 2
NS = 16
NW = NC * NS
C = 128


def _rsqrt16(d):
    i = lax.bitcast_convert_type(d, jnp.int32)
    i = jnp.int32(0x5F3759DF) - jnp.right_shift(i, 1)
    y = lax.bitcast_convert_type(i, jnp.float32)
    hd = 0.5 * d
    for _ in range(3):
        y = y * (1.5 - hd * y * y)
    return y


def _make_deg_kernel(npad, chunks, seg):
    mesh = plsc.VectorSubcoreMesh(core_axis_name="c", subcore_axis_name="s")

    @functools.partial(
        pl.kernel,
        out_type=jax.ShapeDtypeStruct((NC, npad), jnp.float32),
        mesh=mesh,
        scratch_types=[
            pltpu.VMEM((chunks, C), jnp.int32),
            pltpu.VMEM((chunks, C), jnp.float32),
            pltpu.VMEM((seg,), jnp.float32),
            pltpu.VMEM_SHARED((npad,), jnp.float32),
        ],
        compiler_params=pltpu.CompilerParams(needs_layout_passes=False, use_tc_tiling_on_sc=False),
    )
    def deg_kernel(col_hbm, ew_hbm, deg_out, col_v, ew_v, zero_v, deg_sh):
        cid = lax.axis_index("c")
        sid = lax.axis_index("s")
        wid = cid * NS + sid
        pltpu.sync_copy(col_hbm.at[wid], col_v)
        pltpu.sync_copy(ew_hbm.at[wid], ew_v)

        def zbody(i, carry):
            zero_v[pl.ds(i * 16, 16)] = jnp.zeros((16,), jnp.float32)
            return carry

        lax.fori_loop(0, seg // 16, zbody, 0)
        pltpu.sync_copy(zero_v, deg_sh.at[pl.ds(sid * seg, seg)])
        plsc.subcore_barrier()

        def body(j, carry):
            pltpu.sync_copy(ew_v.at[j], deg_sh.at[col_v.at[j]], add=True)
            return carry

        lax.fori_loop(0, chunks, body, 0)
        plsc.subcore_barrier()
        pltpu.sync_copy(deg_sh.at[pl.ds(sid * seg, seg)],
                        deg_out.at[cid, pl.ds(sid * seg, seg)])

    return deg_kernel


def _make_gcn_kernel(npad, chunks, seg, d_out):
    mesh = plsc.VectorSubcoreMesh(core_axis_name="c", subcore_axis_name="s")

    @functools.partial(
        pl.kernel,
        out_type=jax.ShapeDtypeStruct((NC, npad, d_out), jnp.float32),
        mesh=mesh,
        scratch_types=[
            pltpu.VMEM((2, C), jnp.int32),
            pltpu.VMEM((chunks, C), jnp.float32),
            pltpu.VMEM((NC, seg), jnp.float32),
            pltpu.VMEM((npad,), jnp.float32),
            pltpu.VMEM((C, d_out), jnp.float32),
            pltpu.VMEM((C,), jnp.float32),
            pltpu.VMEM_SHARED((npad, d_out), jnp.float32),
        ],
        compiler_params=pltpu.CompilerParams(
            needs_layout_passes=False, use_tc_tiling_on_sc=False),
    )
    def gcn_kernel(rc_hbm, ew_hbm, h_hbm, deg_hbm, s_out,
                   rc_v, ew_v, degb_v, dinv_v, rows_v, coef_v, acc_sh):
        cid = lax.axis_index("c")
        sid = lax.axis_index("s")
        wid = cid * NS + sid
        pltpu.sync_copy(ew_hbm.at[wid], ew_v)

        def dblk(bb, carry):
            pltpu.sync_copy(deg_hbm.at[:, pl.ds(bb * seg, seg)], degb_v)

            def dbody(i, carry2):
                sl = pl.ds(i * 16, 16)
                d = degb_v[0, sl] + degb_v[1, sl]
                dinv_v[pl.ds(bb * seg + i * 16, 16)] = _rsqrt16(d)
                return carry2

            lax.fori_loop(0, seg // 16, dbody, 0)
            return carry

        lax.fori_loop(0, npad // seg, dblk, 0)

        def zbody(r, carry):
            for cb in range(d_out // 16):
                rows_v[r, pl.ds(cb * 16, 16)] = jnp.zeros((16,), jnp.float32)
            return carry

        lax.fori_loop(0, C, zbody, 0)
        for t in range(seg // C):
            pltpu.sync_copy(rows_v, acc_sh.at[pl.ds(sid * seg + t * C, C)])
        plsc.subcore_barrier()

        def chunk(j, carry):
            pltpu.sync_copy(rc_hbm.at[wid, j], rc_v)
            pltpu.sync_copy(h_hbm.at[rc_v.at[0]], rows_v)
            for k in range(C // 16):
                sl = pl.ds(k * 16, 16)
                r16 = rc_v[0, sl]
                c16 = rc_v[1, sl]
                w16 = ew_v[j, sl]
                dr = plsc.load_gather(dinv_v, [r16])
                dc = plsc.load_gather(dinv_v, [c16])
                coef_v[sl] = w16 * dr * dc

            def scale(g, carry2):
                c16 = coef_v[pl.ds(g * 16, 16)]
                for l in range(16):
                    s = c16[l]
                    r = g * 16 + l
                    for cb in range(d_out // 16):
                        sl2 = pl.ds(cb * 16, 16)
                        rows_v[r, sl2] = rows_v[r, sl2] * s
                return carry2

            lax.fori_loop(0, C // 16, scale, 0)
            pltpu.sync_copy(rows_v, acc_sh.at[rc_v.at[1]], add=True)
            return carry

        lax.fori_loop(0, chunks, chunk, 0)
        plsc.subcore_barrier()
        pltpu.sync_copy(acc_sh.at[pl.ds(sid * seg, seg)],
                        s_out.at[cid, pl.ds(sid * seg, seg)])

    return gcn_kernel


def _matmul(x, W):
    n, d_in = x.shape
    d_out = W.shape[1]
    bs = 1000 if n % 1000 == 0 else n

    def body(x_ref, w_ref, o_ref):
        o_ref[...] = jnp.dot(x_ref[...], w_ref[...],
                             preferred_element_type=jnp.float32)

    return pl.pallas_call(
        body,
        grid=(n // bs,),
        in_specs=[
            pl.BlockSpec((bs, d_in), lambda i: (i, 0)),
            pl.BlockSpec((d_in, d_out), lambda i: (0, 0)),
        ],
        out_specs=pl.BlockSpec((bs, d_out), lambda i: (i, 0)),
        out_shape=jax.ShapeDtypeStruct((n, d_out), jnp.float32),
    )(x, W)


def _combine(S, b2):
    _, n, d_out = S.shape
    bs = 1000 if n % 1000 == 0 else n

    def body(s_ref, b_ref, o_ref):
        o_ref[...] = s_ref[0] + s_ref[1] + b_ref[...]

    return pl.pallas_call(
        body,
        grid=(n // bs,),
        in_specs=[
            pl.BlockSpec((NC, bs, d_out), lambda i: (0, i, 0)),
            pl.BlockSpec((1, d_out), lambda i: (0, 0)),
        ],
        out_specs=pl.BlockSpec((bs, d_out), lambda i: (i, 0)),
        out_shape=jax.ShapeDtypeStruct((n, d_out), jnp.float32),
    )(S, b2)


def kernel(x, edge_index, edge_weight, W, b):
    n = x.shape[0]
    e = edge_index.shape[1]
    d_out = W.shape[1]

    row = edge_index[0].astype(jnp.int32)
    col = edge_index[1].astype(jnp.int32)
    loop = jnp.arange(n, dtype=jnp.int32)
    rows = jnp.concatenate([row, loop])
    cols = jnp.concatenate([col, loop])
    ews = jnp.concatenate([edge_weight.astype(jnp.float32),
                           jnp.ones((n,), jnp.float32)])

    per = NW * C
    chunks = -(-(e + n) // per)
    epad = per * chunks
    pad = epad - (e + n)
    rows_p = jnp.concatenate([rows, jnp.zeros((pad,), jnp.int32)])
    cols_p = jnp.concatenate([cols, jnp.zeros((pad,), jnp.int32)])
    ews_p = jnp.concatenate([ews, jnp.zeros((pad,), jnp.float32)])

    seg = ((n + NS * C - 1) // (NS * C)) * C
    npad = NS * seg
    dh = d_out // NC

    deg = _make_deg_kernel(npad, chunks, seg)(
        cols_p.reshape(NW, chunks, C), ews_p.reshape(NW, chunks, C))
    h = _matmul(x, W)
    rc = jnp.stack([rows_p.reshape(NW, chunks, C),
                    cols_p.reshape(NW, chunks, C)], axis=2)
    S = _make_gcn_kernel(npad, chunks, seg, d_out)(
        rc, ews_p.reshape(NW, chunks, C), h, deg)
    out = _combine(S[:, :n, :], b.reshape(1, d_out).astype(jnp.float32))
    return out
